# baseline (speedup 1.0000x reference)
import functools
import math

import jax
import jax.numpy as jnp
from jax.experimental import pallas as pl
from jax.experimental.pallas import tpu as pltpu


def _router_block(x_ref, w_ref, idx_ref, loc_ref, gate_ref, carry_ref,
                  tri_ref, *, bt, num_experts):
    b = pl.program_id(0)

    @pl.when(b == 0)
    def _():
        carry_ref[...] = jnp.zeros_like(carry_ref)
        ti = jax.lax.broadcasted_iota(jnp.int32, (bt, bt), 0)
        tj = jax.lax.broadcasted_iota(jnp.int32, (bt, bt), 1)
        tri_ref[...] = (tj < ti).astype(jnp.float32)

    x = x_ref[...]
    w = w_ref[...]
    logits = jax.lax.dot_general(
        x, w, (((1,), (0,)), ((), ())), preferred_element_type=jnp.float32)

    m = jnp.max(logits, axis=1, keepdims=True)
    eiota = jax.lax.broadcasted_iota(jnp.int32, logits.shape, 1)
    idx = jnp.min(jnp.where(logits == m, eiota, num_experts), axis=1)
    gate = 1.0 / jnp.sum(jnp.exp(logits - m), axis=1)

    maskf = (eiota == idx[:, None]).astype(jnp.float32)
    prev = jax.lax.dot(tri_ref[...], maskf, preferred_element_type=jnp.float32)

    carry = carry_ref[...]
    loc = jnp.sum((prev + carry) * maskf, axis=1)
    carry_ref[...] = carry + jnp.sum(maskf, axis=0, keepdims=True)

    idx_ref[...] = idx.reshape(1, 1, bt)
    loc_ref[...] = loc.astype(jnp.int32).reshape(1, 1, bt)
    gate_ref[...] = gate.reshape(1, 1, bt)


def kernel(input, W):
    tokens, model_dim = input.shape
    num_experts = W.shape[0]
    bt = min(1024, tokens)
    nblk = tokens // bt
    capacity = int(math.ceil(tokens / num_experts))

    body = functools.partial(_router_block, bt=bt, num_experts=num_experts)
    idx, loc, gate = pl.pallas_call(
        body,
        grid=(nblk,),
        in_specs=[
            pl.BlockSpec((bt, model_dim), lambda i: (i, 0)),
            pl.BlockSpec((model_dim, num_experts), lambda i: (0, 0)),
        ],
        out_specs=[
            pl.BlockSpec((1, 1, bt), lambda i: (i, 0, 0)),
            pl.BlockSpec((1, 1, bt), lambda i: (i, 0, 0)),
            pl.BlockSpec((1, 1, bt), lambda i: (i, 0, 0)),
        ],
        out_shape=[
            jax.ShapeDtypeStruct((nblk, 1, bt), jnp.int32),
            jax.ShapeDtypeStruct((nblk, 1, bt), jnp.int32),
            jax.ShapeDtypeStruct((nblk, 1, bt), jnp.float32),
        ],
        scratch_shapes=[pltpu.VMEM((1, num_experts), jnp.float32),
                        pltpu.VMEM((bt, bt), jnp.float32)],
    )(input, W.T)

    return (idx.reshape(tokens), jnp.int32(capacity),
            loc.reshape(tokens), gate.reshape(tokens),
            jnp.int32(num_experts))


# speedup vs baseline: 1.0174x; 1.0174x over previous
import functools
import math

import jax
import jax.numpy as jnp
from jax.experimental import pallas as pl
from jax.experimental.pallas import tpu as pltpu


def _router_block(x_ref, w_ref, idx_ref, loc_ref, gate_ref, carry_ref,
                  tri_ref, *, bt, num_experts):
    b = pl.program_id(0)

    @pl.when(b == 0)
    def _():
        carry_ref[...] = jnp.zeros_like(carry_ref)
        ti = jax.lax.broadcasted_iota(jnp.int32, (bt, bt), 0)
        tj = jax.lax.broadcasted_iota(jnp.int32, (bt, bt), 1)
        tri_ref[...] = (tj < ti).astype(jnp.float32)

    x = x_ref[...]
    w = w_ref[...]
    logits = jax.lax.dot_general(
        x, w, (((1,), (1,)), ((), ())), preferred_element_type=jnp.float32)

    m = jnp.max(logits, axis=1, keepdims=True)
    eiota = jax.lax.broadcasted_iota(jnp.int32, logits.shape, 1)
    idx = jnp.min(jnp.where(logits == m, eiota, num_experts), axis=1)
    gate = 1.0 / jnp.sum(jnp.exp(logits - m), axis=1)

    maskf = (eiota == idx[:, None]).astype(jnp.float32)
    prev = jax.lax.dot(tri_ref[...], maskf, preferred_element_type=jnp.float32)

    carry = carry_ref[...]
    loc = jnp.sum((prev + carry) * maskf, axis=1)
    carry_ref[...] = carry + jnp.sum(maskf, axis=0, keepdims=True)

    idx_ref[...] = idx.reshape(1, 1, bt)
    loc_ref[...] = loc.astype(jnp.int32).reshape(1, 1, bt)
    gate_ref[...] = gate.reshape(1, 1, bt)


def kernel(input, W):
    tokens, model_dim = input.shape
    num_experts = W.shape[0]
    bt = min(1024, tokens)
    nblk = tokens // bt
    capacity = int(math.ceil(tokens / num_experts))

    body = functools.partial(_router_block, bt=bt, num_experts=num_experts)
    idx, loc, gate = pl.pallas_call(
        body,
        grid=(nblk,),
        in_specs=[
            pl.BlockSpec((bt, model_dim), lambda i: (i, 0)),
            pl.BlockSpec((num_experts, model_dim), lambda i: (0, 0)),
        ],
        out_specs=[
            pl.BlockSpec((1, 1, bt), lambda i: (i, 0, 0)),
            pl.BlockSpec((1, 1, bt), lambda i: (i, 0, 0)),
            pl.BlockSpec((1, 1, bt), lambda i: (i, 0, 0)),
        ],
        out_shape=[
            jax.ShapeDtypeStruct((nblk, 1, bt), jnp.int32),
            jax.ShapeDtypeStruct((nblk, 1, bt), jnp.int32),
            jax.ShapeDtypeStruct((nblk, 1, bt), jnp.float32),
        ],
        scratch_shapes=[pltpu.VMEM((1, num_experts), jnp.float32),
                        pltpu.VMEM((bt, bt), jnp.float32)],
    )(input, W)

    return (idx.reshape(tokens), jnp.int32(capacity),
            loc.reshape(tokens), gate.reshape(tokens),
            jnp.int32(num_experts))


# speedup vs baseline: 1.0377x; 1.0199x over previous
import functools
import math

import jax
import jax.numpy as jnp
from jax.experimental import pallas as pl
from jax.experimental.pallas import tpu as pltpu


def _router_block(x_ref, w_ref, idx_ref, loc_ref, gate_ref, carry_ref,
                  tri_ref, *, bt, num_experts):
    b = pl.program_id(0)

    @pl.when(b == 0)
    def _():
        carry_ref[...] = jnp.zeros_like(carry_ref)
        ti = jax.lax.broadcasted_iota(jnp.int32, (bt, bt), 0)
        tj = jax.lax.broadcasted_iota(jnp.int32, (bt, bt), 1)
        tri_ref[...] = (tj < ti).astype(jnp.float32)

    x = x_ref[...]
    w = w_ref[...]
    logits = jax.lax.dot_general(
        x, w, (((1,), (1,)), ((), ())), preferred_element_type=jnp.float32)

    m = jnp.max(logits, axis=1, keepdims=True)
    eiota = jax.lax.broadcasted_iota(jnp.int32, logits.shape, 1)
    idx = jnp.min(jnp.where(logits == m, eiota, num_experts), axis=1)
    gate = 1.0 / jnp.sum(jnp.exp(logits - m), axis=1)

    maskf = (eiota == idx[:, None]).astype(jnp.float32)
    prev = jax.lax.dot(tri_ref[...], maskf, preferred_element_type=jnp.float32)

    carry = carry_ref[...]
    loc = jnp.sum((prev + carry) * maskf, axis=1)
    carry_ref[...] = carry + jnp.sum(maskf, axis=0, keepdims=True)

    idx_ref[...] = idx.reshape(1, 1, bt)
    loc_ref[...] = loc.astype(jnp.int32).reshape(1, 1, bt)
    gate_ref[...] = gate.reshape(1, 1, bt)


def kernel(input, W):
    tokens, model_dim = input.shape
    num_experts = W.shape[0]
    bt = min(512, tokens)
    nblk = tokens // bt
    capacity = int(math.ceil(tokens / num_experts))

    body = functools.partial(_router_block, bt=bt, num_experts=num_experts)
    idx, loc, gate = pl.pallas_call(
        body,
        grid=(nblk,),
        in_specs=[
            pl.BlockSpec((bt, model_dim), lambda i: (i, 0)),
            pl.BlockSpec((num_experts, model_dim), lambda i: (0, 0)),
        ],
        out_specs=[
            pl.BlockSpec((1, 1, bt), lambda i: (i, 0, 0)),
            pl.BlockSpec((1, 1, bt), lambda i: (i, 0, 0)),
            pl.BlockSpec((1, 1, bt), lambda i: (i, 0, 0)),
        ],
        out_shape=[
            jax.ShapeDtypeStruct((nblk, 1, bt), jnp.int32),
            jax.ShapeDtypeStruct((nblk, 1, bt), jnp.int32),
            jax.ShapeDtypeStruct((nblk, 1, bt), jnp.float32),
        ],
        scratch_shapes=[pltpu.VMEM((1, num_experts), jnp.float32),
                        pltpu.VMEM((bt, bt), jnp.float32)],
    )(input, W)

    return (idx.reshape(tokens), jnp.int32(capacity),
            loc.reshape(tokens), gate.reshape(tokens),
            jnp.int32(num_experts))
